# Initial kernel scaffold; baseline (speedup 1.0000x reference)
import jax
import jax.numpy as jnp
from jax.experimental import pallas as pl


def kernel(x, edge_index, batch, edge_attr, We0, be0, W1_0, b1_0, gamma0, beta0, W2_0, b2_0, We1, be1, W1_1, b1_1, gamma1, beta1, W2_1, b2_1, Wfc1, Wfc2):
    raise NotImplementedError("write your pallas kernel here")


# speedup vs baseline: 2.6551x; 2.6551x over previous
import functools

import jax
import jax.numpy as jnp
from jax import lax
from jax.experimental import pallas as pl
from jax.experimental.pallas import tpu as pltpu
from jax.experimental.pallas import tpu_sc as plsc

N = 10000
E = 320000
G = 64
D = 128
ED = 16
HFC = 64
OUT = 4

NC = 2
NS = 16
NW = NC * NS
EPT = E // NW
C = 80
NCHUNK = EPT // C
ZR = 80
STRIPE = 640


def _ea_body(eattr_ref, We0_ref, be0_ref, We1_ref, be1_ref, ea0_ref, ea1_ref):
    a = eattr_ref[...]
    ea0_ref[...] = (
        jnp.dot(a, We0_ref[...], preferred_element_type=jnp.float32) + be0_ref[...]
    )
    ea1_ref[...] = (
        jnp.dot(a, We1_ref[...], preferred_element_type=jnp.float32) + be1_ref[...]
    )


def _compute_ea(edge_attr, We0, be0, We1, be1):
    BLK = 4000
    grid = (E // BLK,)
    return pl.pallas_call(
        _ea_body,
        grid=grid,
        in_specs=[
            pl.BlockSpec((BLK, ED), lambda i: (i, 0)),
            pl.BlockSpec((ED, D), lambda i: (0, 0)),
            pl.BlockSpec((1, D), lambda i: (0, 0)),
            pl.BlockSpec((ED, D), lambda i: (0, 0)),
            pl.BlockSpec((1, D), lambda i: (0, 0)),
        ],
        out_specs=[
            pl.BlockSpec((BLK, D), lambda i: (i, 0)),
            pl.BlockSpec((BLK, D), lambda i: (i, 0)),
        ],
        out_shape=[
            jax.ShapeDtypeStruct((E, D), jnp.float32),
            jax.ShapeDtypeStruct((E, D), jnp.float32),
        ],
    )(edge_attr, We0, be0.reshape(1, D), We1, be1.reshape(1, D))


def _edge_body(x_hbm, src_hbm, dst_hbm, ea_hbm, out_hbm,
               srcv, dstv, xv, eav, zerov, aggs, sem):
    c = lax.axis_index("c")
    s = lax.axis_index("s")

    zeros16 = jnp.zeros((16,), jnp.float32)

    def zrow(i, _):
        r = i // 8
        col = (i % 8) * 16
        zerov[r, pl.ds(col, 16)] = zeros16
        return 0

    lax.fori_loop(0, ZR * 8, zrow, 0)

    r_base = s * STRIPE
    n_cp = jnp.where(s < NS - 1, STRIPE // ZR, (N - (NS - 1) * STRIPE) // ZR)

    def zcopy(k, _):
        pltpu.sync_copy(zerov, aggs.at[pl.ds(r_base + k * ZR, ZR)])
        return 0

    lax.fori_loop(0, n_cp, zcopy, 0)
    plsc.subcore_barrier()

    base = (c * NS + s) * EPT

    def chunk(i, _):
        off = base + i * C
        pltpu.sync_copy(src_hbm.at[pl.ds(off, C)], srcv)
        pltpu.sync_copy(dst_hbm.at[pl.ds(off, C)], dstv)
        pltpu.sync_copy(ea_hbm.at[pl.ds(off, C)], eav)
        pltpu.async_copy(x_hbm.at[srcv], xv, sem).wait()

        def row(r, _):
            for j in range(8):
                sl = pl.ds(j * 16, 16)
                eav[r, sl] = jnp.maximum(eav[r, sl] + xv[r, sl], 0.0)
            return 0

        lax.fori_loop(0, C, row, 0)
        pltpu.sync_copy(eav, aggs.at[dstv], add=True)
        return 0

    lax.fori_loop(0, NCHUNK, chunk, 0)
    plsc.subcore_barrier()

    def wcopy(k, _):
        r0 = r_base + k * ZR
        pltpu.sync_copy(aggs.at[pl.ds(r0, ZR)], zerov)
        pltpu.sync_copy(zerov, out_hbm.at[c].at[pl.ds(r0, ZR)])
        return 0

    lax.fori_loop(0, n_cp, wcopy, 0)


def _edge_pass(x, src, dst, ea):
    mesh = plsc.VectorSubcoreMesh(core_axis_name="c", subcore_axis_name="s",
                                  num_cores=NC, num_subcores=NS)
    k = functools.partial(
        pl.kernel,
        out_type=jax.ShapeDtypeStruct((NC, N, D), jnp.float32),
        mesh=mesh,
        scratch_types=[
            pltpu.VMEM((C,), jnp.int32),
            pltpu.VMEM((C,), jnp.int32),
            pltpu.VMEM((C, D), jnp.float32),
            pltpu.VMEM((C, D), jnp.float32),
            pltpu.VMEM((ZR, D), jnp.float32),
            pltpu.VMEM_SHARED((N, D), jnp.float32),
            pltpu.SemaphoreType.DMA,
        ],
    )(_edge_body)
    return k(x, src, dst, ea)


def _mlp(h, W1_ref, b1_ref, g_ref, bt_ref, W2_ref, b2_ref):
    t = jnp.dot(h, W1_ref[...], preferred_element_type=jnp.float32) + b1_ref[...]
    mu = jnp.mean(t, axis=0, keepdims=True)
    d = t - mu
    var = jnp.mean(d * d, axis=0, keepdims=True)
    t = d * (g_ref[...] * lax.rsqrt(var + 1e-5)) + bt_ref[...]
    t = jnp.maximum(t, 0.0)
    t = jnp.dot(t, W2_ref[...], preferred_element_type=jnp.float32) + b2_ref[...]
    return jnp.maximum(t, 0.0)


def _mlp_body(x_ref, agg_ref, W1_ref, b1_ref, g_ref, bt_ref, W2_ref, b2_ref,
              out_ref):
    h = x_ref[...] + agg_ref[0] + agg_ref[1]
    out_ref[...] = _mlp(h, W1_ref, b1_ref, g_ref, bt_ref, W2_ref, b2_ref)


def _mlp_call(x, agg, W1, b1, g, bt, W2, b2):
    return pl.pallas_call(
        _mlp_body,
        out_shape=jax.ShapeDtypeStruct((N, D), jnp.float32),
    )(x, agg, W1, b1.reshape(1, D), g.reshape(1, D), bt.reshape(1, D),
      W2, b2.reshape(1, D))


def _final_body(x_ref, agg_ref, W1_ref, b1_ref, g_ref, bt_ref, W2_ref, b2_ref,
                Wfc1_ref, Wfc2_ref, batch_ref, out_ref):
    h = x_ref[...] + agg_ref[0] + agg_ref[1]
    h = _mlp(h, W1_ref, b1_ref, g_ref, bt_ref, W2_ref, b2_ref)
    hf = jnp.maximum(jnp.dot(h, Wfc1_ref[...], preferred_element_type=jnp.float32), 0.0)
    lg = jnp.maximum(jnp.dot(hf, Wfc2_ref[...], preferred_element_type=jnp.float32), 0.0)

    b = batch_ref[...]
    gids = lax.broadcasted_iota(jnp.int32, (G, 1), 0)
    cmp = b == gids
    rowmax = jnp.max(lg, axis=1)
    neg = jnp.float32(-3.4e38)
    m = jnp.max(jnp.where(cmp, rowmax[None, :], neg), axis=1)
    m_node = jnp.max(jnp.where(cmp, m[:, None], neg), axis=0)
    e = jnp.exp(lg - m_node[:, None])
    rowsum = jnp.sum(e, axis=1)
    ssum = jnp.sum(jnp.where(cmp, rowsum[None, :], 0.0), axis=1)
    s_node = jnp.sum(jnp.where(cmp, ssum[:, None], 0.0), axis=0)
    out_ref[...] = e / s_node[:, None]


def _final_call(x, agg, W1, b1, g, bt, W2, b2, Wfc1, Wfc2, batch):
    return pl.pallas_call(
        _final_body,
        out_shape=jax.ShapeDtypeStruct((N, OUT), jnp.float32),
    )(x, agg, W1, b1.reshape(1, D), g.reshape(1, D), bt.reshape(1, D),
      W2, b2.reshape(1, D), Wfc1, Wfc2, batch.reshape(1, N))


def kernel(x, edge_index, batch, edge_attr,
           We0, be0, W1_0, b1_0, gamma0, beta0, W2_0, b2_0,
           We1, be1, W1_1, b1_1, gamma1, beta1, W2_1, b2_1,
           Wfc1, Wfc2):
    src = edge_index[0]
    dst = edge_index[1]

    ea0, ea1 = _compute_ea(edge_attr, We0, be0, We1, be1)

    agg0 = _edge_pass(x, src, dst, ea0)
    h1 = _mlp_call(x, agg0, W1_0, b1_0, gamma0, beta0, W2_0, b2_0)

    agg1 = _edge_pass(h1, src, dst, ea1)
    out = _final_call(h1, agg1, W1_1, b1_1, gamma1, beta1, W2_1, b2_1,
                      Wfc1, Wfc2, batch)
    return out.reshape(-1)
